# Initial kernel scaffold; baseline (speedup 1.0000x reference)
import jax
import jax.numpy as jnp
from jax.experimental import pallas as pl


def kernel(x, R, centroids):
    raise NotImplementedError("write your pallas kernel here")


# speedup vs baseline: 2.8877x; 2.8877x over previous
import functools

import jax
import jax.numpy as jnp
from jax.experimental import pallas as pl
from jax.experimental.pallas import tpu as pltpu

_BN = 1024


def _obrq_kernel(x_ref, R_ref, c_ref, out_ref):
    x = x_ref[...]
    R = R_ref[...]
    c = c_ref[...]

    norm = jnp.sqrt(jnp.sum(x * x, axis=-1, keepdims=True))
    xn = x / (norm + 1e-8)

    xr = jax.lax.dot_general(
        xn, R, (((1,), (0,)), ((), ())),
        preferred_element_type=jnp.float32)

    rr = jnp.sum(xr * xr, axis=-1, keepdims=True)
    m = jax.lax.dot_general(
        xr, c, (((1,), (1,)), ((), ())),
        preferred_element_type=jnp.float32)
    cn = jnp.sum(c * c, axis=-1)[None, :]
    d2 = (rr - 2.0 * m) + cn
    sel1 = d2[:, 1:2] < d2[:, 0:1]
    x_mse = jnp.where(sel1, c[1:2, :], c[0:1, :])

    residual = xr - x_mse
    signs = jnp.where(residual >= 0, 1.0, -1.0)
    scale = jnp.mean(jnp.abs(residual), axis=-1, keepdims=True)
    corrected = x_mse + scale * signs

    recon = jax.lax.dot_general(
        corrected, R, (((1,), (1,)), ((), ())),
        preferred_element_type=jnp.float32)
    out_ref[...] = recon * norm


@jax.jit
def kernel(x, R, centroids):
    n, d = x.shape
    grid = (n // _BN,)
    return pl.pallas_call(
        _obrq_kernel,
        grid=grid,
        in_specs=[
            pl.BlockSpec((_BN, d), lambda i: (i, 0)),
            pl.BlockSpec((d, d), lambda i: (0, 0)),
            pl.BlockSpec(centroids.shape, lambda i: (0, 0)),
        ],
        out_specs=pl.BlockSpec((_BN, d), lambda i: (i, 0)),
        out_shape=jax.ShapeDtypeStruct((n, d), jnp.float32),
    )(x, R, centroids)


# speedup vs baseline: 4.8835x; 1.6911x over previous
import functools

import jax
import jax.numpy as jnp
from jax.experimental import pallas as pl
from jax.experimental.pallas import tpu as pltpu

_BN = 1024


def _obrq_kernel(x_ref, R_ref, c_ref, out_ref):
    x = x_ref[...]
    R = R_ref[...]
    c = c_ref[...]

    norm = jnp.sqrt(jnp.sum(x * x, axis=-1, keepdims=True))
    xn = x * (1.0 / (norm + 1e-8))

    xr = jax.lax.dot_general(
        xn, R, (((1,), (0,)), ((), ())),
        preferred_element_type=jnp.float32)

    rr = jnp.sum(xr * xr, axis=-1, keepdims=True)
    m = jax.lax.dot_general(
        xr, c, (((1,), (1,)), ((), ())),
        preferred_element_type=jnp.float32)
    cn = jnp.sum(c * c, axis=-1)[None, :]
    d2 = (rr - 2.0 * m) + cn
    sel1 = d2[:, 1:2] < d2[:, 0:1]
    x_mse = jnp.where(sel1, c[1:2, :], c[0:1, :])

    residual = xr - x_mse
    signs = jnp.where(residual >= 0, 1.0, -1.0)
    scale = jnp.mean(jnp.abs(residual), axis=-1, keepdims=True)

    crot = jax.lax.dot_general(
        c, R, (((1,), (1,)), ((), ())),
        preferred_element_type=jnp.float32)
    srot = jax.lax.dot_general(
        signs.astype(jnp.bfloat16), R.astype(jnp.bfloat16),
        (((1,), (1,)), ((), ())),
        preferred_element_type=jnp.float32)
    x_mse_rot = jnp.where(sel1, crot[1:2, :], crot[0:1, :])
    recon = x_mse_rot + scale * srot
    out_ref[...] = recon * norm


@jax.jit
def kernel(x, R, centroids):
    n, d = x.shape
    grid = (n // _BN,)
    return pl.pallas_call(
        _obrq_kernel,
        grid=grid,
        in_specs=[
            pl.BlockSpec((_BN, d), lambda i: (i, 0)),
            pl.BlockSpec((d, d), lambda i: (0, 0)),
            pl.BlockSpec(centroids.shape, lambda i: (0, 0)),
        ],
        out_specs=pl.BlockSpec((_BN, d), lambda i: (i, 0)),
        out_shape=jax.ShapeDtypeStruct((n, d), jnp.float32),
    )(x, R, centroids)


# speedup vs baseline: 4.9595x; 1.0156x over previous
import functools

import jax
import jax.numpy as jnp
from jax.experimental import pallas as pl
from jax.experimental.pallas import tpu as pltpu

_BN = 1024


def _obrq_kernel(x_ref, R_ref, c_ref, out_ref):
    x = x_ref[...]
    R = R_ref[...]
    c = c_ref[...]

    norm = jnp.sqrt(jnp.sum(x * x, axis=-1, keepdims=True))
    xn = x * (1.0 / (norm + 1e-8))

    xr = jax.lax.dot_general(
        xn, R, (((1,), (0,)), ((), ())),
        preferred_element_type=jnp.float32)

    m = jax.lax.dot_general(
        xr, c, (((1,), (1,)), ((), ())),
        preferred_element_type=jnp.float32)
    cn = jnp.sum(c * c, axis=-1, keepdims=True)
    d2_0 = cn[0, 0] - 2.0 * m[:, 0:1]
    d2_1 = cn[1, 0] - 2.0 * m[:, 1:2]
    sel1 = d2_1 < d2_0
    x_mse = jnp.where(sel1, c[1:2, :], c[0:1, :])

    residual = xr - x_mse
    signs = jnp.where(residual >= 0, 1.0, -1.0).astype(jnp.bfloat16)
    scale = jnp.mean(jnp.abs(residual), axis=-1, keepdims=True)

    crot = jax.lax.dot_general(
        c, R, (((1,), (1,)), ((), ())),
        preferred_element_type=jnp.float32)
    srot = jax.lax.dot_general(
        signs, R.astype(jnp.bfloat16),
        (((1,), (1,)), ((), ())),
        preferred_element_type=jnp.float32)
    x_mse_rot = jnp.where(sel1, crot[1:2, :], crot[0:1, :])
    recon = x_mse_rot + scale * srot
    out_ref[...] = recon * norm


@jax.jit
def kernel(x, R, centroids):
    n, d = x.shape
    grid = (n // _BN,)
    return pl.pallas_call(
        _obrq_kernel,
        grid=grid,
        in_specs=[
            pl.BlockSpec((_BN, d), lambda i: (i, 0)),
            pl.BlockSpec((d, d), lambda i: (0, 0)),
            pl.BlockSpec(centroids.shape, lambda i: (0, 0)),
        ],
        out_specs=pl.BlockSpec((_BN, d), lambda i: (i, 0)),
        out_shape=jax.ShapeDtypeStruct((n, d), jnp.float32),
    )(x, R, centroids)
